# Initial kernel scaffold; baseline (speedup 1.0000x reference)
import jax
import jax.numpy as jnp
from jax.experimental import pallas as pl


def kernel(xyz, features, params):
    raise NotImplementedError("write your pallas kernel here")


# speedup vs baseline: 9.9155x; 9.9155x over previous
import functools
import math

import jax
import jax.numpy as jnp
import numpy as np
from jax import lax
from jax.experimental import pallas as pl
from jax.experimental.pallas import tpu as pltpu

_INTERPRET = False


def _fps_body(npoint, n, n8, x_ref, fidx_ref, cent_ref):
    xyz = x_ref[0]
    x, y, z = xyz[0], xyz[1], xyz[2]
    rowi = lax.broadcasted_iota(jnp.int32, (8, n8), 0)
    coli = lax.broadcasted_iota(jnp.int32, (8, n8), 1)
    pidx = rowi * n8 + coli

    def body(i, carry):
        dist, far = carry
        sel = pidx == far
        cx = jnp.sum(jnp.where(sel, x, 0.0))
        cy = jnp.sum(jnp.where(sel, y, 0.0))
        cz = jnp.sum(jnp.where(sel, z, 0.0))
        d = (x - cx) ** 2 + (y - cy) ** 2 + (z - cz) ** 2
        dist = jnp.minimum(dist, d)
        m = jnp.max(dist)
        far_new = jnp.min(jnp.where(dist == m, pidx, n))
        fidx_ref[0, pl.ds(i, 1), :] = jnp.full((1, 1), far, jnp.int32)
        cvec = jnp.concatenate(
            [jnp.full((1, 1), cx), jnp.full((1, 1), cy),
             jnp.full((1, 1), cz), jnp.zeros((1, 1), jnp.float32)], axis=1)
        cent_ref[0, pl.ds(i, 1), :] = cvec
        return dist, far_new

    init = (jnp.full((8, n8), 1e10, jnp.float32), jnp.int32(0))
    lax.fori_loop(0, npoint, body, init)


def _fps(xyz, npoint):
    B, N, _ = xyz.shape
    n8 = N // 8
    xyz_r = jnp.transpose(xyz, (0, 2, 1)).reshape(B, 3, 8, n8)
    fidx, cent = pl.pallas_call(
        functools.partial(_fps_body, npoint, N, n8),
        grid=(B,),
        in_specs=[pl.BlockSpec((1, 3, 8, n8), lambda b: (b, 0, 0, 0))],
        out_specs=[pl.BlockSpec((1, npoint, 1), lambda b: (b, 0, 0)),
                   pl.BlockSpec((1, npoint, 4), lambda b: (b, 0, 0))],
        out_shape=[jax.ShapeDtypeStruct((B, npoint, 1), jnp.int32),
                   jax.ShapeDtypeStruct((B, npoint, 4), jnp.float32)],
        interpret=_INTERPRET,
    )(xyz_r)
    return fidx[..., 0], cent[..., :3]


def _mm_body(a_ref, bt_ref, o_ref):
    o_ref[0] = jnp.dot(a_ref[0], bt_ref[0],
                       preferred_element_type=jnp.float32)


def _pallas_sqdist(a, b):
    B, M, _ = a.shape
    N = b.shape[1]
    BM = min(M, 128)
    bt = jnp.transpose(b, (0, 2, 1))
    ab = pl.pallas_call(
        _mm_body,
        grid=(B, M // BM),
        in_specs=[pl.BlockSpec((1, BM, 3), lambda i, m: (i, m, 0)),
                  pl.BlockSpec((1, 3, N), lambda i, m: (i, 0, 0))],
        out_specs=pl.BlockSpec((1, BM, N), lambda i, m: (i, m, 0)),
        out_shape=jax.ShapeDtypeStruct((B, M, N), jnp.float32),
        interpret=_INTERPRET,
    )(a, bt)
    aa = jnp.sum(a * a, axis=-1)[:, :, None]
    bb = jnp.sum(b * b, axis=-1)[:, None, :]
    return aa + bb - 2.0 * ab


def _topk_body(n, r1sq, r2sq, ns1, ns2, d_ref, idx1_ref, idx2_ref):
    b = pl.program_id(0)
    D = d_ref[0]
    coli = lax.broadcasted_iota(jnp.int32, D.shape, 1)
    vals, idxs = [], []
    Dw = D
    for _ in range(ns2):
        m = jnp.min(Dw, axis=1, keepdims=True)
        ik = jnp.min(jnp.where(Dw == m, coli, n), axis=1, keepdims=True)
        vals.append(m)
        idxs.append(ik)
        Dw = jnp.where(coli == ik, jnp.inf, Dw)
    dmat = jnp.concatenate(vals, axis=1)
    imat = jnp.concatenate(idxs, axis=1)
    off = b * n
    near = imat[:, :1]
    fb1 = jnp.where(dmat[:, :1] <= r1sq, near, 0)
    fb2 = jnp.where(dmat[:, :1] <= r2sq, near, 0)
    idx2_ref[0] = jnp.where(dmat <= r2sq, imat, fb2) + off
    idx1_ref[0] = jnp.where(dmat[:, :ns1] <= r1sq, imat[:, :ns1], fb1) + off


def _ball_topk(centers, points, r1, r2, ns1, ns2):
    B, M, _ = centers.shape
    N = points.shape[1]
    BM = min(M, 128)
    D = _pallas_sqdist(centers, points)
    idx1, idx2 = pl.pallas_call(
        functools.partial(_topk_body, N, np.float32(r1 * r1),
                          np.float32(r2 * r2), ns1, ns2),
        grid=(B, M // BM),
        in_specs=[pl.BlockSpec((1, BM, N), lambda b, m: (b, m, 0))],
        out_specs=[pl.BlockSpec((1, BM, ns1), lambda b, m: (b, m, 0)),
                   pl.BlockSpec((1, BM, ns2), lambda b, m: (b, m, 0))],
        out_shape=[jax.ShapeDtypeStruct((B, M, ns1), jnp.int32),
                   jax.ShapeDtypeStruct((B, M, ns2), jnp.int32)],
        interpret=_INTERPRET,
    )(D)
    return idx1, idx2


def _top3_body(n2, d_ref, idx_ref, w_ref):
    b = pl.program_id(0)
    D = d_ref[0]
    coli = lax.broadcasted_iota(jnp.int32, D.shape, 1)
    vals, idxs = [], []
    Dw = D
    for _ in range(3):
        m = jnp.min(Dw, axis=1, keepdims=True)
        ik = jnp.min(jnp.where(Dw == m, coli, n2), axis=1, keepdims=True)
        vals.append(m)
        idxs.append(ik)
        Dw = jnp.where(coli == ik, jnp.inf, Dw)
    dmat = jnp.maximum(jnp.concatenate(vals, axis=1), 1e-10)
    w = 1.0 / dmat
    w = w / jnp.sum(w, axis=1, keepdims=True)
    imat = jnp.concatenate(idxs, axis=1) + b * n2
    zi = jnp.zeros_like(imat[:, :1]) + b * n2
    zw = jnp.zeros_like(w[:, :1])
    idx_ref[0] = jnp.concatenate([imat, zi], axis=1)
    w_ref[0] = jnp.concatenate([w, zw], axis=1)


def _three_nn(xyz1, xyz2):
    B, n1, _ = xyz1.shape
    n2 = xyz2.shape[1]
    BN = min(n1, 512)
    D = _pallas_sqdist(xyz1, xyz2)
    idx, w = pl.pallas_call(
        functools.partial(_top3_body, n2),
        grid=(B, n1 // BN),
        in_specs=[pl.BlockSpec((1, BN, n2), lambda b, m: (b, m, 0))],
        out_specs=[pl.BlockSpec((1, BN, 4), lambda b, m: (b, m, 0)),
                   pl.BlockSpec((1, BN, 4), lambda b, m: (b, m, 0))],
        out_shape=[jax.ShapeDtypeStruct((B, n1, 4), jnp.int32),
                   jax.ShapeDtypeStruct((B, n1, 4), jnp.float32)],
        interpret=_INTERPRET,
    )(D)
    return idx, w


def _mlp_body(nlayers, relu_last, x_ref, *refs):
    out_ref = refs[-1]
    h = x_ref[0]
    for i in range(nlayers):
        w = refs[2 * i][...]
        bb = refs[2 * i + 1][...]
        h = jnp.dot(h, w, preferred_element_type=jnp.float32) + bb
        if relu_last or i < nlayers - 1:
            h = jnp.maximum(h, 0.0)
    out_ref[0] = h


def _mlp(x, layers, relu_last=True):
    B, n, _ = x.shape
    BN = min(n, 1024)
    cout = layers[-1][0].shape[1]
    ops = []
    specs = [pl.BlockSpec((1, BN, x.shape[2]), lambda b, m: (b, m, 0))]
    for W, bvec in layers:
        ops += [W, bvec.reshape(1, -1)]
        specs += [pl.BlockSpec(W.shape, lambda b, m: (0, 0)),
                  pl.BlockSpec((1, bvec.shape[0]), lambda b, m: (0, 0))]
    return pl.pallas_call(
        functools.partial(_mlp_body, len(layers), relu_last),
        grid=(B, n // BN),
        in_specs=specs,
        out_specs=pl.BlockSpec((1, BN, cout), lambda b, m: (b, m, 0)),
        out_shape=jax.ShapeDtypeStruct((B, n, cout), jnp.float32),
        interpret=_INTERPRET,
    )(x, *ops)


def _gather_max(table, idx):
    B, M, S = idx.shape
    g = table[idx.reshape(-1)].reshape(B, M, S, -1)
    return jnp.max(g, axis=2)


def _gather_wsum(table, idx, w):
    B, n1, S = idx.shape
    g = table[idx.reshape(-1)].reshape(B, n1, S, -1)
    return jnp.sum(g * w[..., None], axis=2)


def _sa_level(xyz, feats, npoint, radii, nsamples, branches):
    B, N, _ = xyz.shape
    _, centers = _fps(xyz, npoint)
    idx1, idx2 = _ball_topk(centers, xyz, radii[0], radii[1],
                            nsamples[0], nsamples[1])
    outs = []
    for idx_b, layers in zip((idx1, idx2), branches):
        T = _mlp(feats, layers)
        outs.append(_gather_max(T.reshape(B * N, -1), idx_b))
    return centers, jnp.concatenate(outs, axis=-1)


def _fp_level(xyz1, xyz2, feat1, feat2, layers, relu_last=True):
    B, n2, C2 = feat2.shape
    idx, w = _three_nn(xyz1, xyz2)
    interp = _gather_wsum(feat2.reshape(B * n2, C2), idx, w)
    f = jnp.concatenate([interp, feat1], axis=-1)
    return _mlp(f, layers, relu_last=relu_last)


def kernel(xyz, features, params):
    feats = features
    xyz1, f1 = _sa_level(xyz, feats, 1024, [0.05, 0.1], [16, 32], params['sa1'])
    xyz2, f2 = _sa_level(xyz1, f1, 256, [0.1, 0.2], [16, 32], params['sa2'])
    xyz3, f3 = _sa_level(xyz2, f2, 64, [0.2, 0.4], [16, 32], params['sa3'])
    xyz4, f4 = _sa_level(xyz3, f3, 16, [0.4, 0.8], [16, 32], params['sa4'])
    f3 = _fp_level(xyz3, xyz4, f3, f4, params['fp4'])
    f2 = _fp_level(xyz2, xyz3, f2, f3, params['fp3'])
    f1 = _fp_level(xyz1, xyz2, f1, f2, params['fp2'])
    final_layers = list(params['fp1']) + [params['fc1'], params['fc2']]
    f0 = _fp_level(xyz, xyz1, feats, f1, final_layers, relu_last=False)
    return jnp.transpose(f0, (0, 2, 1))


# speedup vs baseline: 11.0266x; 1.1121x over previous
import functools
import math

import jax
import jax.numpy as jnp
import numpy as np
from jax import lax
from jax.experimental import pallas as pl
from jax.experimental.pallas import tpu as pltpu
from jax.experimental.pallas import tpu_sc as plsc

_NW = 32


def _fps_body(npoint, n, n8, x_ref, fidx_ref, cent_ref):
    xyz = x_ref[0]
    x, y, z = xyz[0], xyz[1], xyz[2]
    rowi = lax.broadcasted_iota(jnp.int32, (8, n8), 0)
    coli = lax.broadcasted_iota(jnp.int32, (8, n8), 1)
    pidx = rowi * n8 + coli

    def body(i, carry):
        dist, far = carry
        sel = pidx == far
        cx = jnp.sum(jnp.where(sel, x, 0.0))
        cy = jnp.sum(jnp.where(sel, y, 0.0))
        cz = jnp.sum(jnp.where(sel, z, 0.0))
        d = (x - cx) ** 2 + (y - cy) ** 2 + (z - cz) ** 2
        dist = jnp.minimum(dist, d)
        m = jnp.max(dist)
        far_new = jnp.min(jnp.where(dist == m, pidx, n))
        fidx_ref[0, pl.ds(i, 1), :] = jnp.full((1, 1), far, jnp.int32)
        cvec = jnp.concatenate(
            [jnp.full((1, 1), cx), jnp.full((1, 1), cy),
             jnp.full((1, 1), cz), jnp.zeros((1, 1), jnp.float32)], axis=1)
        cent_ref[0, pl.ds(i, 1), :] = cvec
        return dist, far_new

    init = (jnp.full((8, n8), 1e10, jnp.float32), jnp.int32(0))
    lax.fori_loop(0, npoint, body, init)


def _fps(xyz, npoint):
    B, N, _ = xyz.shape
    n8 = N // 8
    xyz_r = jnp.transpose(xyz, (0, 2, 1)).reshape(B, 3, 8, n8)
    fidx, cent = pl.pallas_call(
        functools.partial(_fps_body, npoint, N, n8),
        grid=(B,),
        in_specs=[pl.BlockSpec((1, 3, 8, n8), lambda b: (b, 0, 0, 0))],
        out_specs=[pl.BlockSpec((1, npoint, 1), lambda b: (b, 0, 0)),
                   pl.BlockSpec((1, npoint, 4), lambda b: (b, 0, 0))],
        out_shape=[jax.ShapeDtypeStruct((B, npoint, 1), jnp.int32),
                   jax.ShapeDtypeStruct((B, npoint, 4), jnp.float32)],
    )(xyz_r)
    return fidx[..., 0], cent[..., :3]


def _mm_body(a_ref, bt_ref, o_ref):
    o_ref[0] = jnp.dot(a_ref[0], bt_ref[0],
                       preferred_element_type=jnp.float32)


def _pallas_sqdist(a, b):
    B, M, _ = a.shape
    N = b.shape[1]
    BM = min(M, 128)
    bt = jnp.transpose(b, (0, 2, 1))
    ab = pl.pallas_call(
        _mm_body,
        grid=(B, M // BM),
        in_specs=[pl.BlockSpec((1, BM, 3), lambda i, m: (i, m, 0)),
                  pl.BlockSpec((1, 3, N), lambda i, m: (i, 0, 0))],
        out_specs=pl.BlockSpec((1, BM, N), lambda i, m: (i, m, 0)),
        out_shape=jax.ShapeDtypeStruct((B, M, N), jnp.float32),
    )(a, bt)
    aa = jnp.sum(a * a, axis=-1)[:, :, None]
    bb = jnp.sum(b * b, axis=-1)[:, None, :]
    return aa + bb - 2.0 * ab


def _topk_body(n, r1sq, r2sq, ns1, ns2, d_ref, idx1_ref, idx2_ref):
    b = pl.program_id(0)
    D = d_ref[0]
    coli = lax.broadcasted_iota(jnp.int32, D.shape, 1)
    vals, idxs = [], []
    Dw = D
    for _ in range(ns2):
        m = jnp.min(Dw, axis=1, keepdims=True)
        ik = jnp.min(jnp.where(Dw == m, coli, n), axis=1, keepdims=True)
        vals.append(m)
        idxs.append(ik)
        Dw = jnp.where(coli == ik, jnp.inf, Dw)
    dmat = jnp.concatenate(vals, axis=1)
    imat = jnp.concatenate(idxs, axis=1)
    off = b * n
    near = imat[:, :1]
    fb1 = jnp.where(dmat[:, :1] <= r1sq, near, 0)
    fb2 = jnp.where(dmat[:, :1] <= r2sq, near, 0)
    idx2_ref[0] = jnp.where(dmat <= r2sq, imat, fb2) + off
    idx1_ref[0] = jnp.where(dmat[:, :ns1] <= r1sq, imat[:, :ns1], fb1) + off


def _ball_topk(centers, points, r1, r2, ns1, ns2):
    B, M, _ = centers.shape
    N = points.shape[1]
    BM = min(M, 128)
    D = _pallas_sqdist(centers, points)
    idx1, idx2 = pl.pallas_call(
        functools.partial(_topk_body, N, np.float32(r1 * r1),
                          np.float32(r2 * r2), ns1, ns2),
        grid=(B, M // BM),
        in_specs=[pl.BlockSpec((1, BM, N), lambda b, m: (b, m, 0))],
        out_specs=[pl.BlockSpec((1, BM, ns1), lambda b, m: (b, m, 0)),
                   pl.BlockSpec((1, BM, ns2), lambda b, m: (b, m, 0))],
        out_shape=[jax.ShapeDtypeStruct((B, M, ns1), jnp.int32),
                   jax.ShapeDtypeStruct((B, M, ns2), jnp.int32)],
    )(D)
    return idx1, idx2


def _top3_body(n2, d_ref, idx_ref, w_ref):
    b = pl.program_id(0)
    D = d_ref[0]
    coli = lax.broadcasted_iota(jnp.int32, D.shape, 1)
    vals, idxs = [], []
    Dw = D
    for _ in range(3):
        m = jnp.min(Dw, axis=1, keepdims=True)
        ik = jnp.min(jnp.where(Dw == m, coli, n2), axis=1, keepdims=True)
        vals.append(m)
        idxs.append(ik)
        Dw = jnp.where(coli == ik, jnp.inf, Dw)
    dmat = jnp.maximum(jnp.concatenate(vals, axis=1), 1e-10)
    w = 1.0 / dmat
    w = w / jnp.sum(w, axis=1, keepdims=True)
    imat = jnp.concatenate(idxs, axis=1) + b * n2
    zi = jnp.zeros_like(imat[:, :1]) + b * n2
    zw = jnp.zeros_like(w[:, :1])
    idx_ref[0] = jnp.concatenate([imat, zi], axis=1)
    w_ref[0] = jnp.concatenate([w, zw], axis=1)


def _three_nn(xyz1, xyz2):
    B, n1, _ = xyz1.shape
    n2 = xyz2.shape[1]
    BN = min(n1, 512)
    D = _pallas_sqdist(xyz1, xyz2)
    idx, w = pl.pallas_call(
        functools.partial(_top3_body, n2),
        grid=(B, n1 // BN),
        in_specs=[pl.BlockSpec((1, BN, n2), lambda b, m: (b, m, 0))],
        out_specs=[pl.BlockSpec((1, BN, 4), lambda b, m: (b, m, 0)),
                   pl.BlockSpec((1, BN, 4), lambda b, m: (b, m, 0))],
        out_shape=[jax.ShapeDtypeStruct((B, n1, 4), jnp.int32),
                   jax.ShapeDtypeStruct((B, n1, 4), jnp.float32)],
    )(D)
    return idx, w


def _mlp_body(nlayers, relu_last, x_ref, *refs):
    out_ref = refs[-1]
    h = x_ref[0]
    for i in range(nlayers):
        w = refs[2 * i][...]
        bb = refs[2 * i + 1][...]
        h = jnp.dot(h, w, preferred_element_type=jnp.float32) + bb
        if relu_last or i < nlayers - 1:
            h = jnp.maximum(h, 0.0)
    out_ref[0] = h


def _mlp(x, layers, relu_last=True):
    B, n, _ = x.shape
    BN = min(n, 1024)
    cout = layers[-1][0].shape[1]
    ops = []
    specs = [pl.BlockSpec((1, BN, x.shape[2]), lambda b, m: (b, m, 0))]
    for W, bvec in layers:
        ops += [W, bvec.reshape(1, -1)]
        specs += [pl.BlockSpec(W.shape, lambda b, m: (0, 0)),
                  pl.BlockSpec((1, bvec.shape[0]), lambda b, m: (0, 0))]
    return pl.pallas_call(
        functools.partial(_mlp_body, len(layers), relu_last),
        grid=(B, n // BN),
        in_specs=specs,
        out_specs=pl.BlockSpec((1, BN, cout), lambda b, m: (b, m, 0)),
        out_shape=jax.ShapeDtypeStruct((B, n, cout), jnp.float32),
    )(x, *ops)


def _sc_gather_max(table, idx):
    G, S = idx.shape
    C = table.shape[1]
    gpw = G // _NW
    cg = max(1, min(gpw, 128 // S))
    nchunk = gpw // cg
    idx_flat = idx.reshape(-1)
    mesh = plsc.VectorSubcoreMesh(core_axis_name="c", subcore_axis_name="s")

    @functools.partial(
        pl.kernel, mesh=mesh,
        out_type=jax.ShapeDtypeStruct((G, C), jnp.float32),
        scratch_types=[pltpu.VMEM((cg * S,), jnp.int32),
                       pltpu.VMEM((cg * S, C), jnp.float32),
                       pltpu.VMEM((cg, C), jnp.float32),
                       pltpu.SemaphoreType.DMA],
    )
    def k(table_hbm, idx_hbm, out_hbm, idx_v, rows_v, out_v, sem):
        wid = lax.axis_index("s") * 2 + lax.axis_index("c")
        base = wid * gpw

        def chunk_body(ci, _):
            gb = base + ci * cg
            pltpu.sync_copy(idx_hbm.at[pl.ds(gb * S, cg * S)], idx_v)
            pltpu.async_copy(table_hbm.at[idx_v], rows_v, sem).wait()

            def g_body(g, _):
                def l_body(l, _):
                    acc = rows_v[g * S, pl.ds(l * 16, 16)]
                    for s2 in range(1, S):
                        acc = jnp.maximum(
                            acc, rows_v[g * S + s2, pl.ds(l * 16, 16)])
                    out_v[g, pl.ds(l * 16, 16)] = acc
                    return 0
                return lax.fori_loop(0, C // 16, l_body, 0)

            lax.fori_loop(0, cg, g_body, 0)
            pltpu.sync_copy(out_v, out_hbm.at[pl.ds(gb, cg)])
            return 0

        lax.fori_loop(0, nchunk, chunk_body, 0)

    return k(table, idx_flat)


def _sc_gather_wsum(table, idx, w):
    G, S = idx.shape
    C = table.shape[1]
    gpw = G // _NW
    cg = max(1, min(gpw, 128 // S))
    nchunk = gpw // cg
    idx_flat = idx.reshape(-1)
    w_flat = jnp.concatenate([w.reshape(-1), jnp.zeros((16,), jnp.float32)])
    mesh = plsc.VectorSubcoreMesh(core_axis_name="c", subcore_axis_name="s")

    @functools.partial(
        pl.kernel, mesh=mesh,
        out_type=jax.ShapeDtypeStruct((G, C), jnp.float32),
        scratch_types=[pltpu.VMEM((cg * S,), jnp.int32),
                       pltpu.VMEM((cg * S + 16,), jnp.float32),
                       pltpu.VMEM((cg * S, C), jnp.float32),
                       pltpu.VMEM((cg, C), jnp.float32),
                       pltpu.SemaphoreType.DMA],
    )
    def k(table_hbm, idx_hbm, w_hbm, out_hbm, idx_v, w_v, rows_v, out_v, sem):
        wid = lax.axis_index("s") * 2 + lax.axis_index("c")
        base = wid * gpw

        def chunk_body(ci, _):
            gb = base + ci * cg
            pltpu.sync_copy(idx_hbm.at[pl.ds(gb * S, cg * S)], idx_v)
            pltpu.sync_copy(w_hbm.at[pl.ds(gb * S, cg * S + 16)], w_v)
            pltpu.async_copy(table_hbm.at[idx_v], rows_v, sem).wait()

            def g_body(g, _):
                wv = w_v[pl.ds(g * S, 16)]

                def l_body(l, _):
                    acc = wv[0] * rows_v[g * S, pl.ds(l * 16, 16)]
                    for s2 in range(1, S):
                        acc = acc + wv[s2] * rows_v[g * S + s2,
                                                    pl.ds(l * 16, 16)]
                    out_v[g, pl.ds(l * 16, 16)] = acc
                    return 0
                return lax.fori_loop(0, C // 16, l_body, 0)

            lax.fori_loop(0, cg, g_body, 0)
            pltpu.sync_copy(out_v, out_hbm.at[pl.ds(gb, cg)])
            return 0

        lax.fori_loop(0, nchunk, chunk_body, 0)

    return k(table, idx_flat, w_flat)


def _gather_max(table, idx):
    B, M, S = idx.shape
    C = table.shape[1]
    return _sc_gather_max(table, idx.reshape(B * M, S)).reshape(B, M, C)


def _gather_wsum(table, idx, w):
    B, n1, S = idx.shape
    C = table.shape[1]
    return _sc_gather_wsum(table, idx.reshape(B * n1, S),
                           w.reshape(B * n1, S)).reshape(B, n1, C)


def _sa_level(xyz, feats, npoint, radii, nsamples, branches):
    B, N, _ = xyz.shape
    _, centers = _fps(xyz, npoint)
    idx1, idx2 = _ball_topk(centers, xyz, radii[0], radii[1],
                            nsamples[0], nsamples[1])
    outs = []
    for idx_b, layers in zip((idx1, idx2), branches):
        cout = layers[-1][0].shape[1]
        if cout % 128:
            Wl, bl = layers[-1]
            pad = 128 - cout
            Wl = jnp.concatenate(
                [Wl, jnp.zeros((Wl.shape[0], pad), Wl.dtype)], axis=1)
            bl = jnp.concatenate([bl, jnp.zeros((pad,), bl.dtype)])
            layers = list(layers[:-1]) + [(Wl, bl)]
        T = _mlp(feats, layers)
        outs.append(_gather_max(T.reshape(B * N, -1), idx_b)[..., :cout])
    return centers, jnp.concatenate(outs, axis=-1)


def _fp_level(xyz1, xyz2, feat1, feat2, layers, relu_last=True):
    B, n2, C2 = feat2.shape
    idx, w = _three_nn(xyz1, xyz2)
    interp = _gather_wsum(feat2.reshape(B * n2, C2), idx, w)
    f = jnp.concatenate([interp, feat1], axis=-1)
    return _mlp(f, layers, relu_last=relu_last)


def kernel(xyz, features, params):
    feats = features
    xyz1, f1 = _sa_level(xyz, feats, 1024, [0.05, 0.1], [16, 32], params['sa1'])
    xyz2, f2 = _sa_level(xyz1, f1, 256, [0.1, 0.2], [16, 32], params['sa2'])
    xyz3, f3 = _sa_level(xyz2, f2, 64, [0.2, 0.4], [16, 32], params['sa3'])
    xyz4, f4 = _sa_level(xyz3, f3, 16, [0.4, 0.8], [16, 32], params['sa4'])
    f3 = _fp_level(xyz3, xyz4, f3, f4, params['fp4'])
    f2 = _fp_level(xyz2, xyz3, f2, f3, params['fp3'])
    f1 = _fp_level(xyz1, xyz2, f1, f2, params['fp2'])
    final_layers = list(params['fp1']) + [params['fc1'], params['fc2']]
    f0 = _fp_level(xyz, xyz1, feats, f1, final_layers, relu_last=False)
    return jnp.transpose(f0, (0, 2, 1))
